# Initial kernel scaffold; baseline (speedup 1.0000x reference)
import jax
import jax.numpy as jnp
from jax.experimental import pallas as pl


def kernel(indices, table):
    raise NotImplementedError("write your pallas kernel here")


# speedup vs baseline: 3.9526x; 3.9526x over previous
import functools

import jax
import jax.numpy as jnp
from jax import lax
from jax.experimental import pallas as pl
from jax.experimental.pallas import tpu as pltpu
from jax.experimental.pallas import tpu_sc as plsc

VOCAB = 100000
EMBED = 64
B = 4096
L = 200
BTOT = B * L

_info = plsc.get_sparse_core_info()
NC, NS = _info.num_cores, _info.num_subcores
NW = NC * NS
PER_W = BTOT // NW
CHUNK = 512
NCHUNK = PER_W // CHUNK

_mesh = plsc.VectorSubcoreMesh(core_axis_name="c", subcore_axis_name="s")


@functools.partial(
    pl.kernel,
    mesh=_mesh,
    out_type=jax.ShapeDtypeStruct((BTOT, EMBED), jnp.float32),
    scratch_types=[
        pltpu.VMEM((CHUNK,), jnp.int32),
        pltpu.VMEM((CHUNK, EMBED), jnp.float32),
        pltpu.SemaphoreType.DMA,
    ],
    compiler_params=pltpu.CompilerParams(use_tc_tiling_on_sc=False),
)
def _gather_kernel(idx_hbm, table_hbm, out_hbm, idx_v, rows_v, sem):
    wid = lax.axis_index("s") * NC + lax.axis_index("c")
    wbase = wid * PER_W

    def step(i, carry):
        base = wbase + i * CHUNK
        pltpu.sync_copy(idx_hbm.at[pl.ds(base, CHUNK)], idx_v)
        pltpu.async_copy(table_hbm.at[idx_v], rows_v, sem).wait()
        pltpu.sync_copy(rows_v, out_hbm.at[pl.ds(base, CHUNK)])
        return carry

    lax.fori_loop(0, NCHUNK, step, 0)


def kernel(indices, table):
    flat_idx = indices.reshape(BTOT).astype(jnp.int32)
    out = _gather_kernel(flat_idx, table)
    return out.reshape(B, L, EMBED)


# speedup vs baseline: 4.2792x; 1.0826x over previous
import functools

import jax
import jax.numpy as jnp
from jax import lax
from jax.experimental import pallas as pl
from jax.experimental.pallas import tpu as pltpu
from jax.experimental.pallas import tpu_sc as plsc

VOCAB = 100000
EMBED = 64
B = 4096
L = 200
BTOT = B * L

_info = plsc.get_sparse_core_info()
NC, NS = _info.num_cores, _info.num_subcores
NW = NC * NS
PER_W = BTOT // NW
CHUNK = 512
NCHUNK = PER_W // CHUNK
NBUF = 2
NG = NCHUNK // NBUF

_mesh = plsc.VectorSubcoreMesh(core_axis_name="c", subcore_axis_name="s")


@functools.partial(
    pl.kernel,
    mesh=_mesh,
    out_type=jax.ShapeDtypeStruct((BTOT, EMBED), jnp.float32),
    scratch_types=[
        pltpu.VMEM((PER_W,), jnp.int32),
        pltpu.VMEM((NBUF, CHUNK, EMBED), jnp.float32),
        pltpu.SemaphoreType.DMA,
        pltpu.SemaphoreType.DMA,
        pltpu.SemaphoreType.DMA,
        pltpu.SemaphoreType.DMA,
    ],
    compiler_params=pltpu.CompilerParams(use_tc_tiling_on_sc=False),
)
def _gather_kernel(idx_hbm, table_hbm, out_hbm, idx_all, rows, gs0, gs1, ws0, ws1):
    gsem = (gs0, gs1)
    wsem = (ws0, ws1)
    wid = lax.axis_index("s") * NC + lax.axis_index("c")
    wbase = wid * PER_W
    pltpu.sync_copy(idx_hbm.at[pl.ds(wbase, PER_W)], idx_all)

    def idx_slice(i):
        return idx_all.at[pl.ds(pl.multiple_of(i * CHUNK, CHUNK), CHUNK)]

    def out_slice(i):
        return out_hbm.at[pl.ds(pl.multiple_of(wbase + i * CHUNK, CHUNK), CHUNK)]

    def fire_gather(i, b):
        pltpu.async_copy(table_hbm.at[idx_slice(i)], rows.at[b], gsem[b])

    def wait_gather(i, b):
        pltpu.make_async_copy(table_hbm.at[idx_slice(i)], rows.at[b], gsem[b]).wait()

    def fire_write(i, b):
        pltpu.async_copy(rows.at[b], out_slice(i), wsem[b])

    def wait_write(i, b):
        pltpu.make_async_copy(rows.at[b], out_slice(i), wsem[b]).wait()

    for b in range(NBUF):
        fire_gather(b, b)

    def group(g, carry):
        for b in range(NBUF):
            i = g * NBUF + b
            wait_gather(i, b)
            fire_write(i, b)
            wait_write(i, b)
            fire_gather(i + NBUF, b)
        return carry

    lax.fori_loop(0, NG - 1, group, 0)

    for b in range(NBUF):
        i = (NG - 1) * NBUF + b
        wait_gather(i, b)
        fire_write(i, b)
    for b in range(NBUF):
        wait_write((NG - 1) * NBUF + b, b)


def kernel(indices, table):
    flat_idx = indices.reshape(BTOT).astype(jnp.int32)
    out = _gather_kernel(flat_idx, table)
    return out.reshape(B, L, EMBED)
